# Initial kernel scaffold; baseline (speedup 1.0000x reference)
import jax
import jax.numpy as jnp
from jax.experimental import pallas as pl


def kernel(token_inp, pos_inp, type_inp, turn_inp, token_table, pos_table, type_table, turn_table):
    raise NotImplementedError("write your pallas kernel here")


# speedup vs baseline: 6.6709x; 6.6709x over previous
import functools

import jax
import jax.numpy as jnp
from jax import lax
from jax.experimental import pallas as pl
from jax.experimental.pallas import tpu as pltpu
from jax.experimental.pallas import tpu_sc as plsc

HIDDEN = 64
B, L = 4096, 200
N = B * L
NC, NS = 2, 16
NW = NC * NS
NPW = N // NW
G = 128
NG = NPW // G
NBUF = 2


def _tc_prep(pos_inp, type_inp, turn_inp, pos_table, type_table, turn_table):
    def body(pi, ti, ui, pt, tt, ut, fused_ref, fidx_ref):
        p = pt[...]
        t = tt[...]
        u = ut[...]
        fused_ref[...] = (p[:, None, None, :] + t[None, :, None, :]
                          + u[None, None, :, :])
        fidx_ref[...] = pi[...] * 32 + ti[...] * 16 + ui[...]

    fused4, fidx = pl.pallas_call(
        body,
        out_shape=[
            jax.ShapeDtypeStruct((512, 2, 16, HIDDEN), jnp.float32),
            jax.ShapeDtypeStruct((B, L), jnp.int32),
        ],
    )(pos_inp, type_inp, turn_inp, pos_table, type_table, turn_table)
    return fused4.reshape(512 * 2 * 16, HIDDEN), fidx


def _sc_embed(token_table, fused_table, tok_idx, fidx):
    mesh = plsc.VectorSubcoreMesh(core_axis_name="c", subcore_axis_name="s")

    @functools.partial(
        pl.kernel,
        out_type=jax.ShapeDtypeStruct((N, HIDDEN), jnp.float32),
        mesh=mesh,
        scratch_types=[
            pltpu.VMEM((NG, G), jnp.int32),
            pltpu.VMEM((NG, G), jnp.int32),
            pltpu.VMEM((NBUF, G, HIDDEN), jnp.float32),
            pltpu.VMEM((NBUF, G, HIDDEN), jnp.float32),
            pltpu.SemaphoreType.DMA,
            pltpu.SemaphoreType.DMA,
        ],
        compiler_params=pltpu.CompilerParams(use_tc_tiling_on_sc=False),
    )
    def kern(tok_tab, fus_tab, tok_i, fus_i, out, idx_t, idx_f, rows_t,
             rows_f, sem0, sem1):
        wid = lax.axis_index("s") * NC + lax.axis_index("c")
        base = wid * NPW
        pltpu.sync_copy(tok_i.at[wid], idx_t)
        pltpu.sync_copy(fus_i.at[wid], idx_f)
        sems = [sem0, sem1]

        def fire(g, b):
            pltpu.make_async_copy(
                tok_tab.at[idx_t.at[g]], rows_t.at[b], sems[b]).start()
            pltpu.make_async_copy(
                fus_tab.at[idx_f.at[g]], rows_f.at[b], sems[b]).start()

        def drain(g, b):
            pltpu.make_async_copy(
                tok_tab.at[idx_t.at[g]], rows_t.at[b], sems[b]).wait()
            pltpu.make_async_copy(
                fus_tab.at[idx_f.at[g]], rows_f.at[b], sems[b]).wait()

        for b in range(NBUF):
            fire(b, b)

        def outer(g0, carry):
            for b in range(NBUF):
                g = g0 * NBUF + b
                drain(g, b)

                def add_row(r, c):
                    for cc in range(HIDDEN // 16):
                        sl = (b, r, pl.ds(cc * 16, 16))
                        plsc.addupdate(rows_t.at[sl], rows_f[sl])
                    return c

                lax.fori_loop(0, G, add_row, carry)
                pltpu.sync_copy(rows_t.at[b],
                                out.at[pl.ds(base + g * G, G)])

                @pl.when(g + NBUF < NG)
                def _():
                    fire(g + NBUF, b)
            return carry

        lax.fori_loop(0, NG // NBUF, outer, 0)

    return kern(token_table, fused_table, tok_idx, fidx)


def kernel(token_inp, pos_inp, type_inp, turn_inp, token_table, pos_table,
           type_table, turn_table):
    fused_table, fidx = _tc_prep(pos_inp, type_inp, turn_inp, pos_table,
                                 type_table, turn_table)
    tok3 = token_inp.astype(jnp.int32).reshape(NW, NG, G)
    fidx3 = fidx.reshape(NW, NG, G)
    out = _sc_embed(token_table, fused_table, tok3, fidx3)
    return out.reshape(B, L, HIDDEN)
